# baseline (speedup 1.0000x reference)
import functools

import jax
import jax.numpy as jnp
from jax import lax
from jax.experimental import pallas as pl
from jax.experimental.pallas import tpu as pltpu
from jax.experimental.pallas import tpu_sc as plsc

_EPS = 1e-10
_LN2 = 0.6931471805599453
_NW = 32
_CHUNK = 8192

_LOG_POLY = (
    0.12485586851835251,
    -0.18030452728271484,
    0.20199713110923767,
    -0.2497013807296753,
    0.33331483602523804,
)


def _flog(x):
    xi = plsc.bitcast(x, jnp.int32)
    k = lax.shift_right_arithmetic(xi - 0x3F3504F3, 23)
    m = plsc.bitcast(xi - lax.shift_left(k, 23), jnp.float32)
    f = m - 1.0
    z = f * f
    p = jnp.float32(_LOG_POLY[0])
    for c in _LOG_POLY[1:]:
        p = p * f + jnp.float32(c)
    return k.astype(jnp.float32) * jnp.float32(_LN2) + (f - 0.5 * z + z * f * p)


def _sc_gumbel(lf, uf):
    total = lf.shape[0]
    per_w = total // _NW
    n_chunks = per_w // _CHUNK
    assert n_chunks % 2 == 0
    mesh = plsc.VectorSubcoreMesh(core_axis_name="c", subcore_axis_name="s")

    @functools.partial(
        pl.kernel,
        mesh=mesh,
        compiler_params=pltpu.CompilerParams(needs_layout_passes=False),
        out_type=[
            jax.ShapeDtypeStruct((total,), jnp.float32),
            jax.ShapeDtypeStruct((total,), jnp.float32),
        ],
        scratch_types=[
            pltpu.VMEM((_CHUNK,), jnp.float32),
            pltpu.VMEM((_CHUNK,), jnp.float32),
            pltpu.VMEM((_CHUNK,), jnp.float32),
            pltpu.VMEM((_CHUNK,), jnp.float32),
            pltpu.VMEM((_CHUNK,), jnp.float32),
            pltpu.VMEM((_CHUNK,), jnp.float32),
            pltpu.VMEM((_CHUNK,), jnp.float32),
            pltpu.VMEM((_CHUNK,), jnp.float32),
            pltpu.SemaphoreType.DMA,
            pltpu.SemaphoreType.DMA,
            pltpu.SemaphoreType.DMA,
            pltpu.SemaphoreType.DMA,
        ],
    )
    def k(l_hbm, u_hbm, h_hbm, y_hbm,
          lA, uA, hA, yA, lB, uB, hB, yB,
          sinA, sinB, soutA, soutB):
        wid = lax.axis_index("s") * 2 + lax.axis_index("c")
        base = wid * per_w
        bufs = ((lA, uA, hA, yA, sinA, soutA), (lB, uB, hB, yB, sinB, soutB))

        def start_in(sl, ci):
            off = base + ci * _CHUNK
            lb_, ub_, _, _, si, _ = bufs[sl]
            pltpu.async_copy(l_hbm.at[pl.ds(off, _CHUNK)], lb_, si)
            pltpu.async_copy(u_hbm.at[pl.ds(off, _CHUNK)], ub_, si)

        def wait_in(sl, ci):
            off = base + ci * _CHUNK
            lb_, ub_, _, _, si, _ = bufs[sl]
            pltpu.make_async_copy(l_hbm.at[pl.ds(off, _CHUNK)], lb_, si).wait()
            pltpu.make_async_copy(u_hbm.at[pl.ds(off, _CHUNK)], ub_, si).wait()

        def start_out(sl, ci):
            off = base + ci * _CHUNK
            _, _, hb_, yb_, _, so = bufs[sl]
            pltpu.async_copy(hb_, h_hbm.at[pl.ds(off, _CHUNK)], so)
            pltpu.async_copy(yb_, y_hbm.at[pl.ds(off, _CHUNK)], so)

        def wait_out(sl, ci):
            off = base + ci * _CHUNK
            _, _, hb_, yb_, _, so = bufs[sl]
            pltpu.make_async_copy(hb_, h_hbm.at[pl.ds(off, _CHUNK)], so).wait()
            pltpu.make_async_copy(yb_, y_hbm.at[pl.ds(off, _CHUNK)], so).wait()

        def compute(sl):
            lbuf, ubuf, hbuf, ybuf, _, _ = bufs[sl]

            def pair_body(pg, c2):
                a0 = (pg // 2) * 256 + (pg % 2) * 64
                for t in range(4):
                    ao = a0 + t * 16
                    bo = ao + 128
                    la = lbuf[pl.ds(ao, 16)]
                    lb = lbuf[pl.ds(bo, 16)]
                    ua = ubuf[pl.ds(ao, 16)]
                    ub = ubuf[pl.ds(bo, 16)]
                    wa = _flog(ua + jnp.float32(_EPS))
                    wb = _flog(ub + jnp.float32(_EPS))
                    ea = jnp.exp((lb - la) * 2.0) * (wa * wa)
                    wb2 = wb * wb
                    ya = wb2 / (wb2 + ea)
                    ha = jnp.where(ea <= wb2, jnp.float32(1.0), jnp.float32(0.0))
                    ybuf[pl.ds(ao, 16)] = ya
                    ybuf[pl.ds(bo, 16)] = 1.0 - ya
                    hbuf[pl.ds(ao, 16)] = ha
                    hbuf[pl.ds(bo, 16)] = 1.0 - ha
                return c2

            lax.fori_loop(0, _CHUNK // 128, pair_body, 0)

        def process(sl, ci):
            wait_in(sl, ci)

            @pl.when(ci >= 2)
            def _():
                wait_out(sl, ci - 2)

            compute(sl)
            start_out(sl, ci)

            @pl.when(ci + 2 < n_chunks)
            def _():
                start_in(sl, ci + 2)

        start_in(0, 0)
        start_in(1, 1)

        def body2(i, carry):
            process(0, 2 * i)
            process(1, 2 * i + 1)
            return carry

        lax.fori_loop(0, n_chunks // 2, body2, 0)
        wait_out(0, n_chunks - 2)
        wait_out(1, n_chunks - 1)

    return k(lf, uf)


def _to_rows(x):
    m, n, two = x.shape
    nb = n // 128
    return x.reshape(m, nb, 128, two).transpose(0, 1, 3, 2).reshape(m * nb * two, 128)


def _from_rows(x, m, n, two):
    nb = n // 128
    return x.reshape(m, nb, two, 128).transpose(0, 1, 3, 2).reshape(m, n, two)


def kernel(logits, u):
    m, n, two = logits.shape
    lf = _to_rows(logits).reshape(-1)
    uf = _to_rows(u).reshape(-1)
    hard, y = _sc_gumbel(lf, uf)
    nrows = m * (n // 128) * two
    return (
        _from_rows(hard.reshape(nrows, 128), m, n, two),
        _from_rows(y.reshape(nrows, 128), m, n, two),
    )


# speedup vs baseline: 2.6856x; 2.6856x over previous
import functools

import jax
import jax.numpy as jnp
from jax import lax
from jax.experimental import pallas as pl
from jax.experimental.pallas import tpu as pltpu
from jax.experimental.pallas import tpu_sc as plsc

_EPS = 1e-10
_LN2 = 0.6931471805599453
_NW = 32
_CHUNK = 8192

_LOG_POLY = (
    0.12485586851835251,
    -0.18030452728271484,
    0.20199713110923767,
    -0.2497013807296753,
    0.33331483602523804,
)


def _flog(x):
    xi = plsc.bitcast(x, jnp.int32)
    k = lax.shift_right_arithmetic(xi - 0x3F3504F3, 23)
    m = plsc.bitcast(xi - lax.shift_left(k, 23), jnp.float32)
    f = m - 1.0
    z = f * f
    p = jnp.float32(_LOG_POLY[0])
    for c in _LOG_POLY[1:]:
        p = p * f + jnp.float32(c)
    return k.astype(jnp.float32) * jnp.float32(_LN2) + (f - 0.5 * z + z * f * p)


def _sc_gumbel(lf, uf):
    total = lf.shape[0]
    per_w = total // _NW
    n_chunks = per_w // _CHUNK
    assert n_chunks % 2 == 0
    mesh = plsc.VectorSubcoreMesh(core_axis_name="c", subcore_axis_name="s")

    @functools.partial(
        pl.kernel,
        mesh=mesh,
        compiler_params=pltpu.CompilerParams(needs_layout_passes=False),
        out_type=[
            jax.ShapeDtypeStruct((total,), jnp.float32),
            jax.ShapeDtypeStruct((total,), jnp.float32),
        ],
        scratch_types=[
            pltpu.VMEM((_CHUNK,), jnp.float32),
            pltpu.VMEM((_CHUNK,), jnp.float32),
            pltpu.VMEM((_CHUNK,), jnp.float32),
            pltpu.VMEM((_CHUNK,), jnp.float32),
            pltpu.VMEM((_CHUNK,), jnp.float32),
            pltpu.VMEM((_CHUNK,), jnp.float32),
            pltpu.VMEM((_CHUNK,), jnp.float32),
            pltpu.VMEM((_CHUNK,), jnp.float32),
            pltpu.SemaphoreType.DMA,
            pltpu.SemaphoreType.DMA,
            pltpu.SemaphoreType.DMA,
            pltpu.SemaphoreType.DMA,
        ],
    )
    def k(l_hbm, u_hbm, h_hbm, y_hbm,
          lA, uA, hA, yA, lB, uB, hB, yB,
          sinA, sinB, soutA, soutB):
        wid = lax.axis_index("s") * 2 + lax.axis_index("c")
        base = wid * per_w
        bufs = ((lA, uA, hA, yA, sinA, soutA), (lB, uB, hB, yB, sinB, soutB))

        def start_in(sl, ci):
            off = base + ci * _CHUNK
            lb_, ub_, _, _, si, _ = bufs[sl]
            pltpu.async_copy(l_hbm.at[pl.ds(off, _CHUNK)], lb_, si)
            pltpu.async_copy(u_hbm.at[pl.ds(off, _CHUNK)], ub_, si)

        def wait_in(sl, ci):
            off = base + ci * _CHUNK
            lb_, ub_, _, _, si, _ = bufs[sl]
            pltpu.make_async_copy(l_hbm.at[pl.ds(off, _CHUNK)], lb_, si).wait()
            pltpu.make_async_copy(u_hbm.at[pl.ds(off, _CHUNK)], ub_, si).wait()

        def start_out(sl, ci):
            off = base + ci * _CHUNK
            _, _, hb_, yb_, _, so = bufs[sl]
            pltpu.async_copy(hb_, h_hbm.at[pl.ds(off, _CHUNK)], so)
            pltpu.async_copy(yb_, y_hbm.at[pl.ds(off, _CHUNK)], so)

        def wait_out(sl, ci):
            off = base + ci * _CHUNK
            _, _, hb_, yb_, _, so = bufs[sl]
            pltpu.make_async_copy(hb_, h_hbm.at[pl.ds(off, _CHUNK)], so).wait()
            pltpu.make_async_copy(yb_, y_hbm.at[pl.ds(off, _CHUNK)], so).wait()

        def compute(sl):
            lbuf, ubuf, hbuf, ybuf, _, _ = bufs[sl]

            def pair_body(pg, c2):
                a0 = pg * 256
                for t in range(8):
                    ao = a0 + t * 16
                    bo = ao + 128
                    la = lbuf[pl.ds(ao, 16)]
                    lb = lbuf[pl.ds(bo, 16)]
                    ua = ubuf[pl.ds(ao, 16)]
                    ub = ubuf[pl.ds(bo, 16)]
                    wa = _flog(ua + jnp.float32(_EPS))
                    wb = _flog(ub + jnp.float32(_EPS))
                    ea = jnp.exp((lb - la) * 2.0) * (wa * wa)
                    wb2 = wb * wb
                    ya = wb2 / (wb2 + ea)
                    ha = jnp.where(ea <= wb2, jnp.float32(1.0), jnp.float32(0.0))
                    ybuf[pl.ds(ao, 16)] = ya
                    ybuf[pl.ds(bo, 16)] = 1.0 - ya
                    hbuf[pl.ds(ao, 16)] = ha
                    hbuf[pl.ds(bo, 16)] = 1.0 - ha
                return c2

            lax.fori_loop(0, _CHUNK // 256, pair_body, 0)

        def process(sl, ci):
            wait_in(sl, ci)

            @pl.when(ci >= 2)
            def _():
                wait_out(sl, ci - 2)

            compute(sl)
            start_out(sl, ci)

            @pl.when(ci + 2 < n_chunks)
            def _():
                start_in(sl, ci + 2)

        start_in(0, 0)
        start_in(1, 1)

        def body2(i, carry):
            process(0, 2 * i)
            process(1, 2 * i + 1)
            return carry

        lax.fori_loop(0, n_chunks // 2, body2, 0)
        wait_out(0, n_chunks - 2)
        wait_out(1, n_chunks - 1)

    return k(lf, uf)


def _to_rows(x):
    m, n, two = x.shape
    nb = n // 128
    return x.reshape(m, nb, 128, two).transpose(0, 1, 3, 2).reshape(m * nb * two, 128)


def _from_rows(x, m, n, two):
    nb = n // 128
    return x.reshape(m, nb, two, 128).transpose(0, 1, 3, 2).reshape(m, n, two)


def kernel(logits, u):
    m, n, two = logits.shape
    lf = _to_rows(logits).reshape(-1)
    uf = _to_rows(u).reshape(-1)
    hard, y = _sc_gumbel(lf, uf)
    nrows = m * (n // 128) * two
    return (
        _from_rows(hard.reshape(nrows, 128), m, n, two),
        _from_rows(y.reshape(nrows, 128), m, n, two),
    )
